# Initial kernel scaffold; baseline (speedup 1.0000x reference)
import jax
import jax.numpy as jnp
from jax.experimental import pallas as pl


def kernel(predicted_boxes, predicted_scores, boxes, prior_boxes):
    raise NotImplementedError("write your pallas kernel here")


# speedup vs baseline: 4.4030x; 4.4030x over previous
import jax
import jax.numpy as jnp
from jax.experimental import pallas as pl
from jax.experimental.pallas import tpu as pltpu

_NP = 20000
_NPAD = 20480
_CHUNK = 2560
_NCH = _NPAD // _CHUNK
_NOBJ = 64
_THR = 0.6


def _body(pr_ref, ox1_ref, oy1_ref, ox2_ref, oy2_ref, bt_ref, pred_ref,
          sc_ref, loc_out, sco_out, colmax, colarg, rval, ridx):
    b = pl.program_id(0)
    ph = pl.program_id(1)
    c = pl.program_id(2)
    nb = pl.num_programs(0)

    first = jnp.logical_and(jnp.logical_and(b == 0, ph == 0), c == 0)

    @pl.when(first)
    def _init():
        loc_out[0, 0] = 0.0
        sco_out[0, 0] = 0.0

    @pl.when(jnp.logical_and(ph == 0, c == 0))
    def _reset():
        rval[...] = jnp.full_like(rval[...], -1.0)
        ridx[...] = jnp.zeros_like(ridx[...])

    glob = c * _CHUNK + jax.lax.broadcasted_iota(jnp.int32, (1, _CHUNK), 1)
    jcol = jax.lax.broadcasted_iota(jnp.int32, (_NOBJ, _CHUNK), 0)

    @pl.when(ph == 0)
    def _phase_a():
        px1 = pr_ref[0:1, :]
        py1 = pr_ref[1:2, :]
        px2 = pr_ref[2:3, :]
        py2 = pr_ref[3:4, :]
        bx1 = ox1_ref[0]
        by1 = oy1_ref[0]
        bx2 = ox2_ref[0]
        by2 = oy2_ref[0]
        iw = jnp.maximum(jnp.minimum(bx2, px2) - jnp.maximum(bx1, px1), 0.0)
        ih = jnp.maximum(jnp.minimum(by2, py2) - jnp.maximum(by1, py1), 0.0)
        inter = iw * ih
        area_o = (bx2 - bx1) * (by2 - by1)
        area_p = (px2 - px1) * (py2 - py1)
        union = jnp.maximum(area_o + area_p - inter, 1e-10)
        iou = inter / union

        cm = jnp.max(iou, axis=0, keepdims=True)
        ca = jnp.min(jnp.where(iou == cm, jcol, _NOBJ), axis=0, keepdims=True)
        colmax[:, pl.ds(c * _CHUNK, _CHUNK)] = cm
        colarg[:, pl.ds(c * _CHUNK, _CHUNK)] = ca

        rm = jnp.max(iou, axis=1, keepdims=True)
        ri = jnp.min(jnp.where(iou == rm, glob, _NPAD), axis=1, keepdims=True)
        upd = rm > rval[...]
        rval[...] = jnp.where(upd, rm, rval[...])
        ridx[...] = jnp.where(upd, ri, ridx[...])

    @pl.when(ph == 1)
    def _phase_b():
        cm = colmax[:, pl.ds(c * _CHUNK, _CHUNK)]
        ca = colarg[:, pl.ds(c * _CHUNK, _CHUNK)]
        pfe = ridx[...]
        match = pfe == glob
        forced = jnp.max(jnp.where(match, 1, 0), axis=0, keepdims=True) > 0
        assigned = jnp.max(jnp.where(match, jcol, -1), axis=0, keepdims=True)
        obj = jnp.where(forced, assigned, ca)

        oh = (jcol == obj).astype(jnp.float32)
        bt = bt_ref[0]
        tl = jax.lax.dot_general(bt, oh, (((1,), (0,)), ((), ())),
                                 preferred_element_type=jnp.float32)
        pred = pred_ref[0]
        valid = glob < _NP
        diff = jnp.where(valid, jnp.abs(pred - tl), 0.0)
        loc_out[0, 0] += jnp.sum(diff)

        @pl.when(b == nb - 1)
        def _score():
            s0 = sc_ref[0:1, :]
            s1 = sc_ref[1:2, :]
            m = jnp.maximum(s0, s1)
            lse = m + jnp.log(jnp.exp(s0 - m) + jnp.exp(s1 - m))
            lbl = jnp.logical_or(forced, cm >= _THR)
            lp = jnp.where(lbl, s1, s0) - lse
            sco_out[0, 0] += jnp.sum(jnp.where(valid, lp, 0.0))


def kernel(predicted_boxes, predicted_scores, boxes, prior_boxes):
    bsz = predicted_boxes.shape[0]
    pad = _NPAD - _NP
    prT = jnp.pad(prior_boxes, ((0, pad), (0, 0))).T
    ox1 = boxes[..., 0:1]
    oy1 = boxes[..., 1:2]
    ox2 = boxes[..., 2:3]
    oy2 = boxes[..., 3:4]
    bT = jnp.swapaxes(boxes, 1, 2)
    predT = jnp.swapaxes(
        jnp.pad(predicted_boxes, ((0, 0), (0, pad), (0, 0))), 1, 2)
    scT = jnp.pad(predicted_scores, ((0, pad), (0, 0))).T

    loc_sum, sco_sum = pl.pallas_call(
        _body,
        grid=(bsz, 2, _NCH),
        in_specs=[
            pl.BlockSpec((4, _CHUNK), lambda b, ph, c: (0, c)),
            pl.BlockSpec((1, _NOBJ, 1), lambda b, ph, c: (b, 0, 0)),
            pl.BlockSpec((1, _NOBJ, 1), lambda b, ph, c: (b, 0, 0)),
            pl.BlockSpec((1, _NOBJ, 1), lambda b, ph, c: (b, 0, 0)),
            pl.BlockSpec((1, _NOBJ, 1), lambda b, ph, c: (b, 0, 0)),
            pl.BlockSpec((1, 4, _NOBJ), lambda b, ph, c: (b, 0, 0)),
            pl.BlockSpec((1, 4, _CHUNK), lambda b, ph, c: (b, 0, c)),
            pl.BlockSpec((2, _CHUNK), lambda b, ph, c: (0, c)),
        ],
        out_specs=[
            pl.BlockSpec((1, 1), lambda b, ph, c: (0, 0),
                         memory_space=pltpu.SMEM),
            pl.BlockSpec((1, 1), lambda b, ph, c: (0, 0),
                         memory_space=pltpu.SMEM),
        ],
        out_shape=[
            jax.ShapeDtypeStruct((1, 1), jnp.float32),
            jax.ShapeDtypeStruct((1, 1), jnp.float32),
        ],
        scratch_shapes=[
            pltpu.VMEM((1, _NPAD), jnp.float32),
            pltpu.VMEM((1, _NPAD), jnp.int32),
            pltpu.VMEM((_NOBJ, 1), jnp.float32),
            pltpu.VMEM((_NOBJ, 1), jnp.int32),
        ],
    )(prT, ox1, oy1, ox2, oy2, bT, predT, scT)

    loc_loss = loc_sum[0, 0] / (bsz * _NP * 4)
    score_loss = -sco_sum[0, 0] / _NP
    return score_loss + loc_loss
